# baseline (speedup 1.0000x reference)
import functools

import jax
import jax.numpy as jnp
from jax import lax
from jax.experimental import pallas as pl
from jax.experimental.pallas import tpu as pltpu
from jax.experimental.pallas import tpu_sc as plsc

_N = 10000
_M = 10000
_E = 320000
_D = 128
_C = 128
_NEG = 0.2
_EPS = 1e-5

_NP = 10240
_NSUB = 16
_EPS_DEN = 1e-16
_E_SUB = _E // _NSUB
_E_WIN = _E_SUB + 96
_NVEC = _E_SUB // 16
_CH = _NP // _NSUB
_CHV = _CH // 16


def _pre_body(ix, iea, ex_, eea, Wi, atti, We, atte,
              gwi, gbi, gmi, gwe, gbe, gme,
              s_all_o, auxi_o, auxe_o):
    def frame(x_ref, ea_ref, W_ref, att_ref, gw_ref, gb_ref, gms_ref,
              s_o, sbase, aux_o):
        x = x_ref[...]
        ea = ea_ref[...]
        W = W_ref[...]
        att = att_ref[...]
        gw = gw_ref[...]
        gb = gb_ref[...]
        gms = gms_ref[...]
        mean = jnp.mean(x, axis=0)
        ex2 = jnp.mean(x * x, axis=0)
        var = ex2 - (2.0 - gms) * gms * mean * mean
        scale = gw / jnp.sqrt(var + _EPS)
        off = gb - scale * gms * mean
        v1 = jnp.dot(att[0], W)
        v2 = jnp.dot(att[1], W)
        zpad = jnp.zeros((_NP - _N,), jnp.float32)
        s_o[pl.ds(sbase, _NP)] = jnp.concatenate(
            [jnp.dot(x, scale * v1), zpad])
        c1 = jnp.sum(off * v1)
        s_o[pl.ds(sbase, _NP)] = s_o[pl.ds(sbase, _NP)] + c1
        s_o[pl.ds(sbase + _NP, _NP)] = jnp.concatenate(
            [jnp.dot(ea, v2), zpad])
        aux_o[0, :] = scale
        aux_o[1, :] = off
        aux_o[2, :] = mean

    frame(ix, iea, Wi, atti, gwi, gbi, gmi, s_all_o, 0, auxi_o)
    frame(ex_, eea, We, atte, gwe, gbe, gme, s_all_o, 2 * _NP, auxe_o)


_pre_call = pl.pallas_call(
    _pre_body,
    out_shape=[
        jax.ShapeDtypeStruct((4 * _NP,), jnp.float32),
        jax.ShapeDtypeStruct((3, _D), jnp.float32),
        jax.ShapeDtypeStruct((3, _D), jnp.float32),
    ],
)


_sc_mesh = plsc.VectorSubcoreMesh(core_axis_name="c", subcore_axis_name="s")


@functools.partial(
    pl.kernel,
    out_type=jax.ShapeDtypeStruct((2 * _NP,), jnp.float32),
    mesh=_sc_mesh,
    compiler_params=pltpu.CompilerParams(needs_layout_passes=False),
    scratch_types=[
        pltpu.VMEM((_NP,), jnp.float32),
        pltpu.VMEM((_NP,), jnp.float32),
        pltpu.VMEM((_NP,), jnp.float32),
        pltpu.VMEM((_NP,), jnp.float32),
        pltpu.VMEM((_E_SUB,), jnp.float32),
        pltpu.VMEM((2, _E_WIN), jnp.int32),
        pltpu.VMEM((_NSUB, _CH), jnp.float32),
        pltpu.VMEM((_CH,), jnp.float32),
        pltpu.VMEM_SHARED((_NSUB, _NP), jnp.float32),
        pltpu.VMEM_SHARED((_NP,), jnp.float32),
    ],
)
def _edge_call(s_all, ei_intra, ei_inter, w_out,
               s1_v, s2_v, den_v, w_v, ex_v, sd_v,
               stage_v, acc_v, stage_sh, sum_sh):
    cid = lax.axis_index("c")
    sid = lax.axis_index("s")
    ebase = sid * _E_SUB
    roff = lax.rem(ebase, 128)
    astart = pl.multiple_of(ebase - roff, 128)

    sbase = pl.multiple_of(cid * (2 * _NP), 8)
    pltpu.sync_copy(s_all.at[pl.ds(sbase, _NP)], s1_v)
    pltpu.sync_copy(s_all.at[pl.ds(sbase + _NP, _NP)], s2_v)
    pltpu.sync_copy(ei_inter.at[:, pl.ds(astart, _E_WIN)], sd_v)

    @pl.when(cid == 0)
    def _():
        pltpu.sync_copy(ei_intra.at[:, pl.ds(astart, _E_WIN)], sd_v)

    zeros16 = jnp.zeros((16,), jnp.float32)

    @plsc.parallel_loop(0, _NP // 16, unroll=8)
    def _zero(i):
        sl = pl.ds(i * 16, 16)
        den_v[sl] = zeros16
        w_v[sl] = zeros16

    @plsc.parallel_loop(0, _NVEC, unroll=10)
    def _bpass(i):
        src = sd_v[0, pl.ds(roff + i * 16, 16)]
        dst = sd_v[1, pl.ds(roff + i * 16, 16)]
        a1 = plsc.load_gather(s1_v, [src])
        a2 = plsc.load_gather(s2_v, [dst])
        al = a1 + a2
        al = jnp.where(al >= 0.0, al, _NEG * al)
        exv = jnp.exp(al)
        ex_v[pl.ds(i * 16, 16)] = exv
        plsc.addupdate_scatter(den_v, [dst], exv)

    def _combine(local_v):
        pltpu.sync_copy(local_v, stage_sh.at[sid])
        plsc.subcore_barrier()
        pltpu.sync_copy(stage_sh.at[:, pl.ds(sid * _CH, _CH)], stage_v)

        @plsc.parallel_loop(0, _CHV, unroll=4)
        def _sum(j):
            sl = pl.ds(j * 16, 16)
            acc = stage_v[0, sl]
            for r in range(1, _NSUB):
                acc = acc + stage_v[r, sl]
            acc_v[sl] = acc

    _combine(den_v)
    pltpu.sync_copy(acc_v, sum_sh.at[pl.ds(sid * _CH, _CH)])
    plsc.subcore_barrier()
    pltpu.sync_copy(sum_sh, den_v)

    @plsc.parallel_loop(0, _NP // 16, unroll=8)
    def _inv(i):
        sl = pl.ds(i * 16, 16)
        den_v[sl] = 1.0 / (den_v[sl] + _EPS_DEN)

    @plsc.parallel_loop(0, _NVEC, unroll=10)
    def _cpass(i):
        src = sd_v[0, pl.ds(roff + i * 16, 16)]
        dst = sd_v[1, pl.ds(roff + i * 16, 16)]
        exv = ex_v[pl.ds(i * 16, 16)]
        dinv = plsc.load_gather(den_v, [dst])
        plsc.addupdate_scatter(w_v, [src], exv * dinv)

    _combine(w_v)
    wbase = pl.multiple_of(cid * _NP + sid * _CH, 8)
    pltpu.sync_copy(acc_v, w_out.at[pl.ds(wbase, _CH)])


def _post_body(ix, ex_, w_ref, auxi, auxe, Wi, bi, We, be, out_ref):
    def frame(x_ref, wbase, aux_ref, W_ref, b_ref):
        x = x_ref[...]
        w2 = w_ref[pl.ds(wbase, _N)].reshape(1, _N)
        scale = aux_ref[0, :]
        off = aux_ref[1, :]
        W = W_ref[...]
        u_raw = jnp.dot(w2, x)
        s_tot = jnp.sum(w2)
        u = scale[None, :] * u_raw + (off * s_tot)[None, :]
        f = lax.dot_general(u, W, (((1,), (1,)), ((), ())))
        return f[0] * (1.0 / _M) + b_ref[...]

    fi = frame(ix, 0, auxi, Wi, bi)
    fe = frame(ex_, _NP, auxe, We, be)
    out_ref[...] = jnp.concatenate(
        [fi, fe, auxi[2, :], auxe[2, :]], axis=0)


_post_call = pl.pallas_call(
    _post_body,
    out_shape=jax.ShapeDtypeStruct((4 * _D,), jnp.float32),
)


def kernel(intra_x, intra_edge_attr, inter_x, inter_edge_attr,
           gn_intra_weight, gn_intra_bias, gn_intra_mean_scale,
           gn_inter_weight, gn_inter_bias, gn_inter_mean_scale,
           W_intra, att_intra, b_intra, W_inter, att_inter, b_inter,
           intra_edge_index, inter_edge_index):
    atti = att_intra.reshape(2, _C)
    atte = att_inter.reshape(2, _C)
    s_all, auxi, auxe = _pre_call(
        intra_x, intra_edge_attr, inter_x, inter_edge_attr,
        W_intra, atti, W_inter, atte,
        gn_intra_weight, gn_intra_bias, gn_intra_mean_scale,
        gn_inter_weight, gn_inter_bias, gn_inter_mean_scale)
    w_all = _edge_call(s_all, intra_edge_index, inter_edge_index)

    return _post_call(intra_x, inter_x, w_all, auxi, auxe,
                      W_intra, b_intra, W_inter, b_inter)


# speedup vs baseline: 1.2622x; 1.2622x over previous
import functools

import jax
import jax.numpy as jnp
from jax import lax
from jax.experimental import pallas as pl
from jax.experimental.pallas import tpu as pltpu
from jax.experimental.pallas import tpu_sc as plsc

_N = 10000
_M = 10000
_E = 320000
_D = 128
_C = 128
_NEG = 0.2
_EPS = 1e-5

_NP = 10240
_NSUB = 16
_EPS_DEN = 1e-16
_E_SUB = _E // _NSUB
_E_WIN = _E_SUB + 96
_NVEC = _E_SUB // 16
_CH = _NP // _NSUB
_CHV = _CH // 16


def _pre_body(ix, iea, ex_, eea, Wi, atti, We, atte,
              gwi, gbi, gmi, gwe, gbe, gme,
              s_all_o, auxi_o, auxe_o):
    def frame(x_ref, ea_ref, W_ref, att_ref, gw_ref, gb_ref, gms_ref,
              s_o, sbase, aux_o):
        x = x_ref[...]
        ea = ea_ref[...]
        W = W_ref[...]
        att = att_ref[...]
        gw = gw_ref[...]
        gb = gb_ref[...]
        gms = gms_ref[...]
        mean = jnp.mean(x, axis=0)
        ex2 = jnp.mean(x * x, axis=0)
        var = ex2 - (2.0 - gms) * gms * mean * mean
        scale = gw / jnp.sqrt(var + _EPS)
        off = gb - scale * gms * mean
        v12 = jnp.dot(att, W)
        v1 = v12[0]
        zpad = jnp.zeros((_NP - _N,), jnp.float32)
        dnums = (((1,), (1,)), ((), ()))
        s1row = lax.dot_general((scale * v1).reshape(1, _D), x, dnums)
        s2row = lax.dot_general(v12[1:2], ea, dnums)
        s_o[pl.ds(sbase, _NP)] = jnp.concatenate([s1row.reshape(_N), zpad])
        c1 = jnp.sum(off * v1)
        s_o[pl.ds(sbase, _NP)] = s_o[pl.ds(sbase, _NP)] + c1
        s_o[pl.ds(sbase + _NP, _NP)] = jnp.concatenate(
            [s2row.reshape(_N), zpad])
        aux_o[0, :] = scale
        aux_o[1, :] = off
        aux_o[2, :] = mean

    frame(ix, iea, Wi, atti, gwi, gbi, gmi, s_all_o, 0, auxi_o)
    frame(ex_, eea, We, atte, gwe, gbe, gme, s_all_o, 2 * _NP, auxe_o)


_pre_call = pl.pallas_call(
    _pre_body,
    out_shape=[
        jax.ShapeDtypeStruct((4 * _NP,), jnp.float32),
        jax.ShapeDtypeStruct((3, _D), jnp.float32),
        jax.ShapeDtypeStruct((3, _D), jnp.float32),
    ],
)


_sc_mesh = plsc.VectorSubcoreMesh(core_axis_name="c", subcore_axis_name="s")


@functools.partial(
    pl.kernel,
    out_type=jax.ShapeDtypeStruct((2 * _NP,), jnp.float32),
    mesh=_sc_mesh,
    compiler_params=pltpu.CompilerParams(needs_layout_passes=False),
    scratch_types=[
        pltpu.VMEM((_NP,), jnp.float32),
        pltpu.VMEM((_NP,), jnp.float32),
        pltpu.VMEM((_NP,), jnp.float32),
        pltpu.VMEM((_NP,), jnp.float32),
        pltpu.VMEM((_E_SUB,), jnp.float32),
        pltpu.VMEM((2, _E_WIN), jnp.int32),
        pltpu.VMEM((_NSUB, _CH), jnp.float32),
        pltpu.VMEM((_CH,), jnp.float32),
        pltpu.VMEM_SHARED((_NSUB, _NP), jnp.float32),
        pltpu.VMEM_SHARED((_NP,), jnp.float32),
    ],
)
def _edge_call(s_all, ei_intra, ei_inter, w_out,
               s1_v, s2_v, den_v, w_v, ex_v, sd_v,
               stage_v, acc_v, stage_sh, sum_sh):
    cid = lax.axis_index("c")
    sid = lax.axis_index("s")
    ebase = sid * _E_SUB
    roff = lax.rem(ebase, 128)
    astart = pl.multiple_of(ebase - roff, 128)

    sbase = pl.multiple_of(cid * (2 * _NP), 8)
    pltpu.sync_copy(s_all.at[pl.ds(sbase, _NP)], s1_v)
    pltpu.sync_copy(s_all.at[pl.ds(sbase + _NP, _NP)], s2_v)
    pltpu.sync_copy(ei_inter.at[:, pl.ds(astart, _E_WIN)], sd_v)

    @pl.when(cid == 0)
    def _():
        pltpu.sync_copy(ei_intra.at[:, pl.ds(astart, _E_WIN)], sd_v)

    zeros16 = jnp.zeros((16,), jnp.float32)

    @plsc.parallel_loop(0, _NP // 16, unroll=8)
    def _zero(i):
        sl = pl.ds(i * 16, 16)
        den_v[sl] = zeros16
        w_v[sl] = zeros16

    @plsc.parallel_loop(0, _NVEC, unroll=10)
    def _bpass(i):
        src = sd_v[0, pl.ds(roff + i * 16, 16)]
        dst = sd_v[1, pl.ds(roff + i * 16, 16)]
        a1 = plsc.load_gather(s1_v, [src])
        a2 = plsc.load_gather(s2_v, [dst])
        al = a1 + a2
        al = jnp.where(al >= 0.0, al, _NEG * al)
        exv = jnp.exp(al)
        ex_v[pl.ds(i * 16, 16)] = exv
        plsc.addupdate_scatter(den_v, [dst], exv)

    def _combine(local_v):
        pltpu.sync_copy(local_v, stage_sh.at[sid])
        plsc.subcore_barrier()
        pltpu.sync_copy(stage_sh.at[:, pl.ds(sid * _CH, _CH)], stage_v)

        @plsc.parallel_loop(0, _CHV, unroll=4)
        def _sum(j):
            sl = pl.ds(j * 16, 16)
            acc = stage_v[0, sl]
            for r in range(1, _NSUB):
                acc = acc + stage_v[r, sl]
            acc_v[sl] = acc

    _combine(den_v)
    pltpu.sync_copy(acc_v, sum_sh.at[pl.ds(sid * _CH, _CH)])
    plsc.subcore_barrier()
    pltpu.sync_copy(sum_sh, den_v)

    @plsc.parallel_loop(0, _NP // 16, unroll=8)
    def _inv(i):
        sl = pl.ds(i * 16, 16)
        den_v[sl] = 1.0 / (den_v[sl] + _EPS_DEN)

    @plsc.parallel_loop(0, _NVEC, unroll=10)
    def _cpass(i):
        src = sd_v[0, pl.ds(roff + i * 16, 16)]
        dst = sd_v[1, pl.ds(roff + i * 16, 16)]
        exv = ex_v[pl.ds(i * 16, 16)]
        dinv = plsc.load_gather(den_v, [dst])
        plsc.addupdate_scatter(w_v, [src], exv * dinv)

    _combine(w_v)
    wbase = pl.multiple_of(cid * _NP + sid * _CH, 8)
    pltpu.sync_copy(acc_v, w_out.at[pl.ds(wbase, _CH)])


def _post_body(ix, ex_, w_ref, auxi, auxe, Wi, bi, We, be, out_ref):
    def frame(x_ref, wbase, aux_ref, W_ref, b_ref):
        x = x_ref[...]
        w2 = w_ref[pl.ds(wbase, _N)].reshape(1, _N)
        scale = aux_ref[0, :]
        off = aux_ref[1, :]
        W = W_ref[...]
        u_raw = jnp.dot(w2, x)
        s_tot = jnp.sum(w2)
        u = scale[None, :] * u_raw + (off * s_tot)[None, :]
        f = lax.dot_general(u, W, (((1,), (1,)), ((), ())))
        return f[0] * (1.0 / _M) + b_ref[...]

    fi = frame(ix, 0, auxi, Wi, bi)
    fe = frame(ex_, _NP, auxe, We, be)
    out_ref[...] = jnp.concatenate(
        [fi, fe, auxi[2, :], auxe[2, :]], axis=0)


_post_call = pl.pallas_call(
    _post_body,
    out_shape=jax.ShapeDtypeStruct((4 * _D,), jnp.float32),
)


def kernel(intra_x, intra_edge_attr, inter_x, inter_edge_attr,
           gn_intra_weight, gn_intra_bias, gn_intra_mean_scale,
           gn_inter_weight, gn_inter_bias, gn_inter_mean_scale,
           W_intra, att_intra, b_intra, W_inter, att_inter, b_inter,
           intra_edge_index, inter_edge_index):
    atti = att_intra.reshape(2, _C)
    atte = att_inter.reshape(2, _C)
    s_all, auxi, auxe = _pre_call(
        intra_x, intra_edge_attr, inter_x, inter_edge_attr,
        W_intra, atti, W_inter, atte,
        gn_intra_weight, gn_intra_bias, gn_intra_mean_scale,
        gn_inter_weight, gn_inter_bias, gn_inter_mean_scale)
    w_all = _edge_call(s_all, intra_edge_index, inter_edge_index)

    return _post_call(intra_x, inter_x, w_all, auxi, auxe,
                      W_intra, b_intra, W_inter, b_inter)


# speedup vs baseline: 1.2635x; 1.0010x over previous
import functools

import jax
import jax.numpy as jnp
from jax import lax
from jax.experimental import pallas as pl
from jax.experimental.pallas import tpu as pltpu
from jax.experimental.pallas import tpu_sc as plsc

_N = 10000
_M = 10000
_E = 320000
_D = 128
_C = 128
_NEG = 0.2
_EPS = 1e-5

_NP = 10240
_NSUB = 16
_EPS_DEN = 1e-16
_E_SUB = _E // _NSUB
_E_WIN = _E_SUB + 96
_NVEC = _E_SUB // 16
_CH = _NP // _NSUB
_CHV = _CH // 16


def _pre_body(ix, iea, ex_, eea, Wi, atti, We, atte,
              gwi, gbi, gmi, gwe, gbe, gme,
              s_all_o, auxi_o, auxe_o):
    def frame(x_ref, ea_ref, W_ref, att_ref, gw_ref, gb_ref, gms_ref,
              s_o, sbase, aux_o):
        x = x_ref[...]
        ea = ea_ref[...]
        W = W_ref[...]
        att = att_ref[...]
        gw = gw_ref[...]
        gb = gb_ref[...]
        gms = gms_ref[...]
        mean = jnp.mean(x, axis=0)
        ex2 = jnp.mean(x * x, axis=0)
        var = ex2 - (2.0 - gms) * gms * mean * mean
        scale = gw / jnp.sqrt(var + _EPS)
        off = gb - scale * gms * mean
        v12 = jnp.dot(att, W)
        v1 = v12[0]
        zpad = jnp.zeros((_NP - _N,), jnp.float32)
        dnums = (((1,), (1,)), ((), ()))
        s1row = lax.dot_general((scale * v1).reshape(1, _D), x, dnums)
        s2row = lax.dot_general(v12[1:2], ea, dnums)
        s_o[pl.ds(sbase, _NP)] = jnp.concatenate([s1row.reshape(_N), zpad])
        c1 = jnp.sum(off * v1)
        s_o[pl.ds(sbase, _NP)] = s_o[pl.ds(sbase, _NP)] + c1
        s_o[pl.ds(sbase + _NP, _NP)] = jnp.concatenate(
            [s2row.reshape(_N), zpad])
        aux_o[0, :] = scale
        aux_o[1, :] = off
        aux_o[2, :] = mean

    frame(ix, iea, Wi, atti, gwi, gbi, gmi, s_all_o, 0, auxi_o)
    frame(ex_, eea, We, atte, gwe, gbe, gme, s_all_o, 2 * _NP, auxe_o)


_pre_call = pl.pallas_call(
    _pre_body,
    out_shape=[
        jax.ShapeDtypeStruct((4 * _NP,), jnp.float32),
        jax.ShapeDtypeStruct((3, _D), jnp.float32),
        jax.ShapeDtypeStruct((3, _D), jnp.float32),
    ],
)


_sc_mesh = plsc.VectorSubcoreMesh(core_axis_name="c", subcore_axis_name="s")


@functools.partial(
    pl.kernel,
    out_type=jax.ShapeDtypeStruct((2 * _NP,), jnp.float32),
    mesh=_sc_mesh,
    compiler_params=pltpu.CompilerParams(needs_layout_passes=False),
    scratch_types=[
        pltpu.VMEM((_NP,), jnp.float32),
        pltpu.VMEM((_NP,), jnp.float32),
        pltpu.VMEM((_NP,), jnp.float32),
        pltpu.VMEM((_NP,), jnp.float32),
        pltpu.VMEM((_E_SUB,), jnp.float32),
        pltpu.VMEM((2, _E_WIN), jnp.int32),
        pltpu.VMEM((_NSUB, _CH), jnp.float32),
        pltpu.VMEM((_CH,), jnp.float32),
        pltpu.VMEM_SHARED((_NSUB, _NP), jnp.float32),
        pltpu.VMEM_SHARED((_NP,), jnp.float32),
    ],
)
def _edge_call(s_all, ei_intra, ei_inter, w_out,
               s1_v, s2_v, den_v, w_v, ex_v, sd_v,
               stage_v, acc_v, stage_sh, sum_sh):
    cid = lax.axis_index("c")
    sid = lax.axis_index("s")
    ebase = sid * _E_SUB
    roff = lax.rem(ebase, 128)
    astart = pl.multiple_of(ebase - roff, 128)

    sc0 = jax.named_scope("sc_init")
    sc0.__enter__()
    sbase = pl.multiple_of(cid * (2 * _NP), 8)
    pltpu.sync_copy(s_all.at[pl.ds(sbase, _NP)], s1_v)
    pltpu.sync_copy(s_all.at[pl.ds(sbase + _NP, _NP)], s2_v)
    pltpu.sync_copy(ei_inter.at[:, pl.ds(astart, _E_WIN)], sd_v)

    @pl.when(cid == 0)
    def _():
        pltpu.sync_copy(ei_intra.at[:, pl.ds(astart, _E_WIN)], sd_v)

    zeros16 = jnp.zeros((16,), jnp.float32)

    @plsc.parallel_loop(0, _NP // 16, unroll=8)
    def _zero(i):
        sl = pl.ds(i * 16, 16)
        den_v[sl] = zeros16
        w_v[sl] = zeros16

    sc0.__exit__(None, None, None)
    scb = jax.named_scope("sc_passB")
    scb.__enter__()

    @plsc.parallel_loop(0, _NVEC, unroll=10)
    def _bpass(i):
        src = sd_v[0, pl.ds(roff + i * 16, 16)]
        dst = sd_v[1, pl.ds(roff + i * 16, 16)]
        a1 = plsc.load_gather(s1_v, [src])
        a2 = plsc.load_gather(s2_v, [dst])
        al = a1 + a2
        al = jnp.where(al >= 0.0, al, _NEG * al)
        exv = jnp.exp(al)
        ex_v[pl.ds(i * 16, 16)] = exv
        plsc.addupdate_scatter(den_v, [dst], exv)

    def _combine(local_v):
        pltpu.sync_copy(local_v, stage_sh.at[sid])
        plsc.subcore_barrier()
        pltpu.sync_copy(stage_sh.at[:, pl.ds(sid * _CH, _CH)], stage_v)

        @plsc.parallel_loop(0, _CHV, unroll=4)
        def _sum(j):
            sl = pl.ds(j * 16, 16)
            acc = stage_v[0, sl]
            for r in range(1, _NSUB):
                acc = acc + stage_v[r, sl]
            acc_v[sl] = acc

    scb.__exit__(None, None, None)
    scc = jax.named_scope("sc_combine_den")
    scc.__enter__()
    _combine(den_v)
    pltpu.sync_copy(acc_v, sum_sh.at[pl.ds(sid * _CH, _CH)])
    plsc.subcore_barrier()
    pltpu.sync_copy(sum_sh, den_v)
    scc.__exit__(None, None, None)
    scd = jax.named_scope("sc_passC")
    scd.__enter__()

    @plsc.parallel_loop(0, _NP // 16, unroll=8)
    def _inv(i):
        sl = pl.ds(i * 16, 16)
        den_v[sl] = 1.0 / (den_v[sl] + _EPS_DEN)

    @plsc.parallel_loop(0, _NVEC, unroll=10)
    def _cpass(i):
        src = sd_v[0, pl.ds(roff + i * 16, 16)]
        dst = sd_v[1, pl.ds(roff + i * 16, 16)]
        exv = ex_v[pl.ds(i * 16, 16)]
        dinv = plsc.load_gather(den_v, [dst])
        plsc.addupdate_scatter(w_v, [src], exv * dinv)

    scd.__exit__(None, None, None)
    sce = jax.named_scope("sc_combine_w")
    sce.__enter__()
    _combine(w_v)
    wbase = pl.multiple_of(cid * _NP + sid * _CH, 8)
    pltpu.sync_copy(acc_v, w_out.at[pl.ds(wbase, _CH)])
    sce.__exit__(None, None, None)


def _post_body(ix, ex_, w_ref, auxi, auxe, Wi, bi, We, be, out_ref):
    def frame(x_ref, wbase, aux_ref, W_ref, b_ref):
        x = x_ref[...]
        w2 = w_ref[pl.ds(wbase, _N)].reshape(1, _N)
        scale = aux_ref[0, :]
        off = aux_ref[1, :]
        W = W_ref[...]
        u_raw = jnp.dot(w2, x)
        s_tot = jnp.sum(w2)
        u = scale[None, :] * u_raw + (off * s_tot)[None, :]
        f = lax.dot_general(u, W, (((1,), (1,)), ((), ())))
        return f[0] * (1.0 / _M) + b_ref[...]

    fi = frame(ix, 0, auxi, Wi, bi)
    fe = frame(ex_, _NP, auxe, We, be)
    out_ref[...] = jnp.concatenate(
        [fi, fe, auxi[2, :], auxe[2, :]], axis=0)


_post_call = pl.pallas_call(
    _post_body,
    out_shape=jax.ShapeDtypeStruct((4 * _D,), jnp.float32),
)


def kernel(intra_x, intra_edge_attr, inter_x, inter_edge_attr,
           gn_intra_weight, gn_intra_bias, gn_intra_mean_scale,
           gn_inter_weight, gn_inter_bias, gn_inter_mean_scale,
           W_intra, att_intra, b_intra, W_inter, att_inter, b_inter,
           intra_edge_index, inter_edge_index):
    atti = att_intra.reshape(2, _C)
    atte = att_inter.reshape(2, _C)
    s_all, auxi, auxe = _pre_call(
        intra_x, intra_edge_attr, inter_x, inter_edge_attr,
        W_intra, atti, W_inter, atte,
        gn_intra_weight, gn_intra_bias, gn_intra_mean_scale,
        gn_inter_weight, gn_inter_bias, gn_inter_mean_scale)
    w_all = _edge_call(s_all, intra_edge_index, inter_edge_index)

    return _post_call(intra_x, inter_x, w_all, auxi, auxe,
                      W_intra, b_intra, W_inter, b_inter)


# speedup vs baseline: 1.3764x; 1.0894x over previous
import functools

import jax
import jax.numpy as jnp
from jax import lax
from jax.experimental import pallas as pl
from jax.experimental.pallas import tpu as pltpu
from jax.experimental.pallas import tpu_sc as plsc

_N = 10000
_M = 10000
_E = 320000
_D = 128
_C = 128
_NEG = 0.2
_EPS = 1e-5

_NP = 10240
_NSUB = 16
_EPS_DEN = 1e-16
_E_SUB = _E // _NSUB
_E_WIN = _E_SUB + 96
_NVEC = _E_SUB // 16
_CH = _NP // _NSUB
_CHV = _CH // 16


def _pre_body(ix, iea, ex_, eea, Wi, atti, We, atte,
              gwi, gbi, gmi, gwe, gbe, gme,
              s_all_o, auxi_o, auxe_o):
    def frame(x_ref, ea_ref, W_ref, att_ref, gw_ref, gb_ref, gms_ref,
              s_o, sbase, aux_o):
        x = x_ref[...]
        ea = ea_ref[...]
        W = W_ref[...]
        att = att_ref[...]
        gw = gw_ref[...]
        gb = gb_ref[...]
        gms = gms_ref[...]
        mean = jnp.mean(x, axis=0)
        ex2 = jnp.mean(x * x, axis=0)
        var = ex2 - (2.0 - gms) * gms * mean * mean
        scale = gw / jnp.sqrt(var + _EPS)
        off = gb - scale * gms * mean
        v12 = jnp.dot(att, W)
        v1 = v12[0]
        zpad = jnp.zeros((_NP - _N,), jnp.float32)
        dnums = (((1,), (1,)), ((), ()))
        s1row = lax.dot_general((scale * v1).reshape(1, _D), x, dnums)
        s2row = lax.dot_general(v12[1:2], ea, dnums)
        s_o[pl.ds(sbase, _NP)] = jnp.concatenate([s1row.reshape(_N), zpad])
        c1 = jnp.sum(off * v1)
        s_o[pl.ds(sbase, _NP)] = s_o[pl.ds(sbase, _NP)] + c1
        s_o[pl.ds(sbase + _NP, _NP)] = jnp.concatenate(
            [s2row.reshape(_N), zpad])
        aux_o[0, :] = scale
        aux_o[1, :] = off
        aux_o[2, :] = mean

    frame(ix, iea, Wi, atti, gwi, gbi, gmi, s_all_o, 0, auxi_o)
    frame(ex_, eea, We, atte, gwe, gbe, gme, s_all_o, 2 * _NP, auxe_o)


_pre_call = pl.pallas_call(
    _pre_body,
    out_shape=[
        jax.ShapeDtypeStruct((4 * _NP,), jnp.float32),
        jax.ShapeDtypeStruct((3, _D), jnp.float32),
        jax.ShapeDtypeStruct((3, _D), jnp.float32),
    ],
)


_sc_mesh = plsc.VectorSubcoreMesh(core_axis_name="c", subcore_axis_name="s")


@functools.partial(
    pl.kernel,
    out_type=jax.ShapeDtypeStruct((2 * _NP,), jnp.float32),
    mesh=_sc_mesh,
    compiler_params=pltpu.CompilerParams(needs_layout_passes=False),
    scratch_types=[
        pltpu.VMEM((_NP,), jnp.float32),
        pltpu.VMEM((_NP,), jnp.float32),
        pltpu.VMEM((_NP,), jnp.float32),
        pltpu.VMEM((_NP,), jnp.float32),
        pltpu.VMEM((_E_SUB,), jnp.float32),
        pltpu.VMEM((2, _E_WIN), jnp.int32),
        pltpu.VMEM((_NSUB, _CH), jnp.float32),
        pltpu.VMEM((_CH,), jnp.float32),
        pltpu.VMEM_SHARED((_NSUB, _NP), jnp.float32),
        pltpu.VMEM_SHARED((_NP,), jnp.float32),
        pltpu.SemaphoreType.DMA,
    ],
)
def _edge_call(s_all, ei_intra, ei_inter, w_out,
               s1_v, s2_v, den_v, w_v, ex_v, sd_v,
               stage_v, acc_v, stage_sh, sum_sh, esem):
    cid = lax.axis_index("c")
    sid = lax.axis_index("s")
    ebase = sid * _E_SUB
    roff = lax.rem(ebase, 128)
    astart = pl.multiple_of(ebase - roff, 128)

    sc0 = jax.named_scope("sc_init")
    sc0.__enter__()
    sbase = pl.multiple_of(cid * (2 * _NP), 8)

    @pl.when(cid == 0)
    def _():
        pltpu.async_copy(ei_intra.at[:, pl.ds(astart, _E_WIN)], sd_v, esem)

    @pl.when(cid != 0)
    def _():
        pltpu.async_copy(ei_inter.at[:, pl.ds(astart, 128)],
                         sd_v.at[:, pl.ds(0, 128)], esem)
        pltpu.async_copy(ei_inter.at[:, pl.ds(astart + 128, _E_WIN - 128)],
                         sd_v.at[:, pl.ds(128, _E_WIN - 128)], esem)

    @pl.when(sid == 0)
    def _():
        pltpu.sync_copy(s_all.at[pl.ds(sbase, _NP)], stage_sh.at[0])
        pltpu.sync_copy(s_all.at[pl.ds(sbase + _NP, _NP)], stage_sh.at[1])

    plsc.subcore_barrier()
    pltpu.sync_copy(stage_sh.at[0], s1_v)
    pltpu.sync_copy(stage_sh.at[1], s2_v)
    plsc.subcore_barrier()

    zeros16 = jnp.zeros((16,), jnp.float32)

    @plsc.parallel_loop(0, _NP // 16, unroll=8)
    def _zero(i):
        sl = pl.ds(i * 16, 16)
        den_v[sl] = zeros16
        w_v[sl] = zeros16

    pltpu.make_async_copy(
        ei_inter.at[:, pl.ds(astart, _E_WIN)], sd_v, esem).wait()

    sc0.__exit__(None, None, None)
    scb = jax.named_scope("sc_passB")
    scb.__enter__()

    @plsc.parallel_loop(0, _NVEC, unroll=10)
    def _bpass(i):
        src = sd_v[0, pl.ds(roff + i * 16, 16)]
        dst = sd_v[1, pl.ds(roff + i * 16, 16)]
        a1 = plsc.load_gather(s1_v, [src])
        a2 = plsc.load_gather(s2_v, [dst])
        al = a1 + a2
        al = jnp.where(al >= 0.0, al, _NEG * al)
        exv = jnp.exp(al)
        ex_v[pl.ds(i * 16, 16)] = exv
        plsc.addupdate_scatter(den_v, [dst], exv)

    def _combine(local_v):
        pltpu.sync_copy(local_v, stage_sh.at[sid])
        plsc.subcore_barrier()
        pltpu.sync_copy(stage_sh.at[:, pl.ds(sid * _CH, _CH)], stage_v)

        @plsc.parallel_loop(0, _CHV, unroll=4)
        def _sum(j):
            sl = pl.ds(j * 16, 16)
            acc = stage_v[0, sl]
            for r in range(1, _NSUB):
                acc = acc + stage_v[r, sl]
            acc_v[sl] = acc

    scb.__exit__(None, None, None)
    scc = jax.named_scope("sc_combine_den")
    scc.__enter__()
    _combine(den_v)
    pltpu.sync_copy(acc_v, sum_sh.at[pl.ds(sid * _CH, _CH)])
    plsc.subcore_barrier()
    pltpu.sync_copy(sum_sh, den_v)
    scc.__exit__(None, None, None)
    scd = jax.named_scope("sc_passC")
    scd.__enter__()

    @plsc.parallel_loop(0, _NP // 16, unroll=8)
    def _inv(i):
        sl = pl.ds(i * 16, 16)
        den_v[sl] = 1.0 / (den_v[sl] + _EPS_DEN)

    @plsc.parallel_loop(0, _NVEC, unroll=10)
    def _cpass(i):
        src = sd_v[0, pl.ds(roff + i * 16, 16)]
        dst = sd_v[1, pl.ds(roff + i * 16, 16)]
        exv = ex_v[pl.ds(i * 16, 16)]
        dinv = plsc.load_gather(den_v, [dst])
        plsc.addupdate_scatter(w_v, [src], exv * dinv)

    scd.__exit__(None, None, None)
    sce = jax.named_scope("sc_combine_w")
    sce.__enter__()
    _combine(w_v)
    wbase = pl.multiple_of(cid * _NP + sid * _CH, 8)
    pltpu.sync_copy(acc_v, w_out.at[pl.ds(wbase, _CH)])
    sce.__exit__(None, None, None)


def _post_body(ix, ex_, w_ref, auxi, auxe, Wi, bi, We, be, out_ref):
    def frame(x_ref, wbase, aux_ref, W_ref, b_ref):
        x = x_ref[...]
        w2 = w_ref[pl.ds(wbase, _N)].reshape(1, _N)
        scale = aux_ref[0, :]
        off = aux_ref[1, :]
        W = W_ref[...]
        u_raw = jnp.dot(w2, x)
        s_tot = jnp.sum(w2)
        u = scale[None, :] * u_raw + (off * s_tot)[None, :]
        f = lax.dot_general(u, W, (((1,), (1,)), ((), ())))
        return f[0] * (1.0 / _M) + b_ref[...]

    fi = frame(ix, 0, auxi, Wi, bi)
    fe = frame(ex_, _NP, auxe, We, be)
    out_ref[...] = jnp.concatenate(
        [fi, fe, auxi[2, :], auxe[2, :]], axis=0)


_post_call = pl.pallas_call(
    _post_body,
    out_shape=jax.ShapeDtypeStruct((4 * _D,), jnp.float32),
)


def kernel(intra_x, intra_edge_attr, inter_x, inter_edge_attr,
           gn_intra_weight, gn_intra_bias, gn_intra_mean_scale,
           gn_inter_weight, gn_inter_bias, gn_inter_mean_scale,
           W_intra, att_intra, b_intra, W_inter, att_inter, b_inter,
           intra_edge_index, inter_edge_index):
    atti = att_intra.reshape(2, _C)
    atte = att_inter.reshape(2, _C)
    s_all, auxi, auxe = _pre_call(
        intra_x, intra_edge_attr, inter_x, inter_edge_attr,
        W_intra, atti, W_inter, atte,
        gn_intra_weight, gn_intra_bias, gn_intra_mean_scale,
        gn_inter_weight, gn_inter_bias, gn_inter_mean_scale)
    w_all = _edge_call(s_all, intra_edge_index, inter_edge_index)

    return _post_call(intra_x, inter_x, w_all, auxi, auxe,
                      W_intra, b_intra, W_inter, b_inter)
